# baseline (speedup 1.0000x reference)
import functools

import jax
import jax.numpy as jnp
from jax import lax
from jax.experimental import pallas as pl
from jax.experimental.pallas import tpu as pltpu
from jax.experimental.pallas import tpu_sc as plsc

L = 16


@functools.lru_cache(maxsize=None)
def _make_pair_gather(B, V2, D):
  info = plsc.get_sparse_core_info()
  NC = info.num_cores
  NW = NC * info.num_subcores
  CH = 128
  DP = 2 * D
  BW = B // NW
  n_ch = BW // CH
  assert B % (CH * NW) == 0 and D % L == 0

  mesh = plsc.VectorSubcoreMesh(core_axis_name="c", subcore_axis_name="s")

  @functools.partial(
      pl.kernel,
      mesh=mesh,
      out_type=jax.ShapeDtypeStruct((B * D,), jnp.float32),
      scratch_types=[
          pltpu.VMEM((BW,), jnp.int32),
          pltpu.VMEM((BW,), jnp.int32),
          pltpu.VMEM((BW, DP), jnp.float32),
          pltpu.VMEM((BW * D,), jnp.float32),
          pltpu.SemaphoreType.DMA,
      ],
  )
  def k(idx_hbm, t2_hbm, out_hbm, idx_raw, rowid, rows_v, out_v, sem):
    wid = lax.axis_index("s") * NC + lax.axis_index("c")
    base = wid * BW
    pltpu.sync_copy(idx_hbm.at[pl.ds(base, BW)], idx_raw)

    for i in range(BW // L):
      rowid[pl.ds(i * L, L)] = idx_raw[pl.ds(i * L, L)] >> 1

    copies = [
        pltpu.async_copy(
            t2_hbm.at[rowid.at[pl.ds(kk * CH, CH)]],
            rows_v.at[pl.ds(kk * CH, CH), :], sem)
        for kk in range(n_ch)
    ]
    for c in copies:
      c.wait()

    def ext_body(i, _):
      v16 = idx_raw[pl.ds(i * L, L)] & 1
      for l in range(L):
        r = i * L + l
        off = v16[l] * D
        for s in range(D // L):
          w = rows_v[r, pl.ds(off + s * L, L)]
          out_v[pl.ds(r * D + s * L, L)] = w
      return _

    lax.fori_loop(0, BW // L, ext_body, jnp.int32(0))
    pltpu.sync_copy(out_v, out_hbm.at[pl.ds(base * D, BW * D)])

  return k


def kernel(num_group, table):
  B, = num_group.shape
  V, D = table.shape
  t2 = table.reshape(V // 2, 2 * D)
  out_flat = _make_pair_gather(B, V // 2, D)(num_group.astype(jnp.int32), t2)
  return out_flat.reshape(B, D)
